# baseline (speedup 1.0000x reference)
import jax
import jax.numpy as jnp
from jax.experimental import pallas as pl
from jax.experimental.pallas import tpu as pltpu

_BLOCK_ROWS = 200


def _gcn_body(adj_ref, x_ref, w_ref, out_ref):
    tmp = jax.lax.dot_general(
        adj_ref[...], x_ref[...],
        (((1,), (0,)), ((), ())),
        preferred_element_type=jnp.float32,
        precision=jax.lax.Precision.HIGHEST,
    )
    out_ref[...] = jax.lax.dot_general(
        tmp, w_ref[...],
        (((1,), (0,)), ((), ())),
        preferred_element_type=jnp.float32,
        precision=jax.lax.Precision.HIGHEST,
    )


def kernel(x, adj, weight):
    n_nodes, f_in = x.shape
    f_out = weight.shape[1]
    br = _BLOCK_ROWS
    grid = (n_nodes + br - 1) // br
    return pl.pallas_call(
        _gcn_body,
        grid=(grid,),
        in_specs=[
            pl.BlockSpec((br, n_nodes), lambda i: (i, 0)),
            pl.BlockSpec((n_nodes, f_in), lambda i: (0, 0)),
            pl.BlockSpec((f_in, f_out), lambda i: (0, 0)),
        ],
        out_specs=pl.BlockSpec((br, f_out), lambda i: (i, 0)),
        out_shape=jax.ShapeDtypeStruct((n_nodes, f_out), jnp.float32),
        compiler_params=pltpu.CompilerParams(
            dimension_semantics=("arbitrary",),
        ),
    )(adj, x, weight)


# speedup vs baseline: 2.7593x; 2.7593x over previous
import jax
import jax.numpy as jnp
from jax.experimental import pallas as pl
from jax.experimental.pallas import tpu as pltpu

_BLOCK_ROWS = 200


def _gcn_body(adj_ref, x_ref, w_ref, out_ref):
    tmp = jax.lax.dot_general(
        adj_ref[...].astype(jnp.bfloat16), x_ref[...],
        (((1,), (0,)), ((), ())),
        preferred_element_type=jnp.float32,
    )
    out_ref[...] = jax.lax.dot_general(
        tmp.astype(jnp.bfloat16), w_ref[...],
        (((1,), (0,)), ((), ())),
        preferred_element_type=jnp.float32,
    )


def kernel(x, adj, weight):
    n_nodes, f_in = x.shape
    f_out = weight.shape[1]
    x = x.astype(jnp.bfloat16)
    weight = weight.astype(jnp.bfloat16)
    br = _BLOCK_ROWS
    grid = (n_nodes + br - 1) // br
    return pl.pallas_call(
        _gcn_body,
        grid=(grid,),
        in_specs=[
            pl.BlockSpec((br, n_nodes), lambda i: (i, 0)),
            pl.BlockSpec((n_nodes, f_in), lambda i: (0, 0)),
            pl.BlockSpec((f_in, f_out), lambda i: (0, 0)),
        ],
        out_specs=pl.BlockSpec((br, f_out), lambda i: (i, 0)),
        out_shape=jax.ShapeDtypeStruct((n_nodes, f_out), jnp.float32),
        compiler_params=pltpu.CompilerParams(
            dimension_semantics=("arbitrary",),
        ),
    )(adj, x, weight)


# speedup vs baseline: 2.8691x; 1.0398x over previous
import jax
import jax.numpy as jnp
from jax.experimental import pallas as pl
from jax.experimental.pallas import tpu as pltpu

_BLOCK_ROWS = 200


def _gcn_body(adj_ref, x_ref, w_ref, out_ref):
    tmp = jax.lax.dot_general(
        adj_ref[...], x_ref[...],
        (((1,), (0,)), ((), ())),
        preferred_element_type=jnp.float32,
    )
    out_ref[...] = jax.lax.dot_general(
        tmp, w_ref[...],
        (((1,), (0,)), ((), ())),
        preferred_element_type=jnp.float32,
    )


def kernel(x, adj, weight):
    n_nodes, f_in = x.shape
    f_out = weight.shape[1]
    br = _BLOCK_ROWS
    grid = (n_nodes + br - 1) // br
    return pl.pallas_call(
        _gcn_body,
        grid=(grid,),
        in_specs=[
            pl.BlockSpec((br, n_nodes), lambda i: (i, 0)),
            pl.BlockSpec((n_nodes, f_in), lambda i: (0, 0)),
            pl.BlockSpec((f_in, f_out), lambda i: (0, 0)),
        ],
        out_specs=pl.BlockSpec((br, f_out), lambda i: (i, 0)),
        out_shape=jax.ShapeDtypeStruct((n_nodes, f_out), jnp.float32),
        compiler_params=pltpu.CompilerParams(
            dimension_semantics=("arbitrary",),
        ),
    )(adj, x, weight)


# speedup vs baseline: 2.9252x; 1.0195x over previous
import jax
import jax.numpy as jnp
from jax.experimental import pallas as pl
from jax.experimental.pallas import tpu as pltpu

_BLOCK_ROWS = 400


def _gcn_body(adj_ref, x_ref, w_ref, out_ref):
    tmp = jax.lax.dot_general(
        adj_ref[...], x_ref[...],
        (((1,), (0,)), ((), ())),
        preferred_element_type=jnp.float32,
    )
    out_ref[...] = jax.lax.dot_general(
        tmp, w_ref[...],
        (((1,), (0,)), ((), ())),
        preferred_element_type=jnp.float32,
    )


def kernel(x, adj, weight):
    n_nodes, f_in = x.shape
    f_out = weight.shape[1]
    br = _BLOCK_ROWS
    grid = (n_nodes + br - 1) // br
    return pl.pallas_call(
        _gcn_body,
        grid=(grid,),
        in_specs=[
            pl.BlockSpec((br, n_nodes), lambda i: (i, 0)),
            pl.BlockSpec((n_nodes, f_in), lambda i: (0, 0)),
            pl.BlockSpec((f_in, f_out), lambda i: (0, 0)),
        ],
        out_specs=pl.BlockSpec((br, f_out), lambda i: (i, 0)),
        out_shape=jax.ShapeDtypeStruct((n_nodes, f_out), jnp.float32),
        compiler_params=pltpu.CompilerParams(
            dimension_semantics=("arbitrary",),
        ),
    )(adj, x, weight)
